# Initial kernel scaffold; baseline (speedup 1.0000x reference)
import jax
import jax.numpy as jnp
from jax.experimental import pallas as pl


def kernel(xyz, xyz_goal, gripper_eulers, params):
    raise NotImplementedError("write your pallas kernel here")


# speedup vs baseline: 1.4568x; 1.4568x over previous
import functools

import jax
import jax.numpy as jnp
from jax.experimental import pallas as pl
from jax.experimental.pallas import tpu as pltpu

_F32 = jnp.float32


def _fps_body(S, xyz_ref, idx_ref, dist_ref):
    G, _, N = xyz_ref.shape
    x = xyz_ref[:, 0, :]
    y = xyz_ref[:, 1, :]
    z = xyz_ref[:, 2, :]
    lane = jax.lax.broadcasted_iota(jnp.int32, (G, N), 1)
    s_lane = jax.lax.broadcasted_iota(jnp.int32, (G, S), 1)
    dist_ref[...] = jnp.full((G, N), 1e10, _F32)

    def step(i, far):
        idx_ref[...] = jnp.where(s_lane == i, far, idx_ref[...])
        m = lane == far
        cx = jnp.sum(jnp.where(m, x, 0.0), axis=1, keepdims=True)
        cy = jnp.sum(jnp.where(m, y, 0.0), axis=1, keepdims=True)
        cz = jnp.sum(jnp.where(m, z, 0.0), axis=1, keepdims=True)
        dx = x - cx
        dy = y - cy
        dz = z - cz
        d = dx * dx + dy * dy + dz * dz
        dist = jnp.minimum(dist_ref[...], d)
        dist_ref[...] = dist
        maxv = jnp.max(dist, axis=1, keepdims=True)
        nxt = jnp.min(jnp.where(dist == maxv, lane, N), axis=1, keepdims=True)
        return nxt.astype(jnp.int32)

    jax.lax.fori_loop(0, S, step, jnp.zeros((G, 1), jnp.int32))


def _fps_pallas_raw(xyz_cn, S):
    B, _, N = xyz_cn.shape
    return pl.pallas_call(
        functools.partial(_fps_body, S),
        out_shape=jax.ShapeDtypeStruct((B, S), jnp.int32),
        scratch_shapes=[pltpu.VMEM((B, N), _F32)],
    )(xyz_cn)


def _fps_xla(xyz, npoint):
    b, n, _ = xyz.shape

    def step(carry, _):
        distance, farthest = carry
        centroid = jnp.take_along_axis(xyz, farthest[:, None, None], axis=1)
        dist = jnp.sum((xyz - centroid) ** 2, axis=-1)
        distance = jnp.minimum(distance, dist)
        nxt = jnp.argmax(distance, axis=-1).astype(jnp.int32)
        return (distance, nxt), farthest

    init = (jnp.full((b, n), 1e10, dtype=xyz.dtype),
            jnp.zeros((b,), dtype=jnp.int32))
    _, idxs = jax.lax.scan(step, init, None, length=npoint)
    return jnp.transpose(idxs, (1, 0))


def _square_distance(src, dst):
    d = -2.0 * jnp.einsum('bmc,bnc->bmn', src, dst)
    d = d + jnp.sum(src ** 2, axis=-1)[:, :, None]
    d = d + jnp.sum(dst ** 2, axis=-1)[:, None, :]
    return d


def _compute_density(xyz, bandwidth):
    sqrdists = _square_distance(xyz, xyz)
    g = jnp.exp(-sqrdists / (2.0 * bandwidth * bandwidth)) / (2.5 * bandwidth)
    return jnp.mean(g, axis=-1)


def _knn_point(nsample, xyz, new_xyz):
    sqrdists = _square_distance(new_xyz, xyz)
    _, idx = jax.lax.top_k(-sqrdists, nsample)
    return idx


def _index_points(points, idx):
    bidx = jnp.arange(points.shape[0])[:, None, None]
    return points[bidx, idx]


def _batchnorm(x, g, b, axes, eps=1e-5):
    m = jnp.mean(x, axis=axes, keepdims=True)
    v = jnp.var(x, axis=axes, keepdims=True)
    return (x - m) / jnp.sqrt(v + eps) * g + b


def _groupnorm(x, g, b, eps=1e-5):
    m = jnp.mean(x, axis=-1, keepdims=True)
    v = jnp.var(x, axis=-1, keepdims=True)
    return (x - m) / jnp.sqrt(v + eps) * g + b


def _sa_tail(p, new_points, grouped_xyz_norm, grouped_density, b, S):
    h = jnp.einsum('bskc,oc->bsko', new_points, p['conv_w']) + p['conv_b']
    h = jax.nn.relu(_batchnorm(h, p['bn_g'], p['bn_b'], (0, 1, 2)))
    inv_max = jnp.max(grouped_density, axis=2, keepdims=True)
    h = h * (grouped_density / inv_max)
    w = grouped_xyz_norm
    for i in (1, 2, 3):
        w = jnp.einsum('bskc,oc->bsko', w, p['wn%d_w' % i]) + p['wn%d_b' % i]
        w = jax.nn.relu(_batchnorm(w, p['wn%d_g' % i], p['wn%d_b2' % i], (0, 1, 2)))
    feat = jnp.einsum('bskc,bskw->bscw', h, w).reshape(b, S, -1)
    feat = feat @ p['lin_w'].T + p['lin_b']
    feat = jax.nn.relu(_batchnorm(feat, p['bnl_g'], p['bnl_b'], (0, 1)))
    return feat


def _sa_group(xyz, points, fps_idx, nsample, bandwidth):
    density = _compute_density(xyz, bandwidth)
    inv_density = 1.0 / density
    new_xyz = jnp.take_along_axis(xyz, fps_idx[:, :, None], axis=1)
    idx = _knn_point(nsample, xyz, new_xyz)
    grouped_xyz = _index_points(xyz, idx)
    grouped_xyz_norm = grouped_xyz - new_xyz[:, :, None, :]
    grouped_points = _index_points(points, idx)
    new_points = jnp.concatenate([grouped_xyz_norm, grouped_points], axis=-1)
    grouped_density = _index_points(inv_density[:, :, None], idx)
    return new_xyz, new_points, grouped_xyz_norm, grouped_density


def _sa_forward(p, xyz, points, npoint, nsample, bandwidth, fps_idx):
    b = xyz.shape[0]
    new_xyz, new_points, gxn, gd = _sa_group(
        xyz, points, fps_idx, nsample, bandwidth)
    feat = _sa_tail(p, new_points, gxn, gd, b, npoint)
    return new_xyz, feat


def _sa_forward_group_all(p, xyz, points, bandwidth):
    b = xyz.shape[0]
    density = _compute_density(xyz, bandwidth)
    inv_density = 1.0 / density
    grouped_xyz_norm = xyz[:, None, :, :]
    new_points = jnp.concatenate(
        [grouped_xyz_norm, points[:, None, :, :]], axis=-1)
    grouped_density = inv_density[:, None, :, None]
    feat = _sa_tail(p, new_points, grouped_xyz_norm, grouped_density, b, 1)
    return feat


def _rot6d(o):
    x_raw = o[:, 0:3]
    y_raw = o[:, 3:6]

    def normalize(v):
        mag = jnp.sqrt(jnp.sum(v ** 2, axis=1, keepdims=True))
        return v / jnp.maximum(mag, 1e-8)

    x = normalize(x_raw)
    z = normalize(jnp.cross(x, y_raw))
    y = jnp.cross(z, x)
    return jnp.stack([x, y, z], axis=2)


def kernel(xyz, xyz_goal, gripper_eulers, params):
    b = xyz.shape[0]
    l0_xyz = jnp.transpose(xyz[:, :3, :], (0, 2, 1))
    l0_points = jnp.transpose(xyz, (0, 2, 1))
    g0 = jnp.transpose(xyz_goal, (0, 2, 1))

    l1x, l1p = _sa_forward(params['sa1'], l0_xyz, l0_points, 512, 32, 0.1,
                           _fps_pallas_raw(xyz, 512))
    g1x, g1p = _sa_forward(params['sa1_g'], g0, g0, 512, 32, 0.1,
                           _fps_pallas_raw(xyz_goal, 512))

    l2x, l2p = _sa_forward(params['sa2'], l1x, l1p, 128, 64, 0.2,
                           _fps_xla(l1x, 128))
    g2x, g2p = _sa_forward(params['sa2_g'], g1x, g1p, 128, 64, 0.2,
                           _fps_xla(g1x, 128))

    l3p = _sa_forward_group_all(params['sa3'], l2x, l2p, 0.4)
    g3p = _sa_forward_group_all(params['sa3_g'], g2x, g2p, 0.4)
    x = l3p.reshape(b, 256)
    g = g3p.reshape(b, 256)

    gr = jax.nn.relu(_groupnorm(
        gripper_eulers @ params['fc1g_w'].T + params['fc1g_b'],
        params['gn1g_g'], params['gn1g_b']))
    gr = jax.nn.relu(_groupnorm(
        gr @ params['fc2g_w'].T + params['fc2g_b'],
        params['gn2g_g'], params['gn2g_b']))
    h = jnp.concatenate([x, g, gr], axis=-1)
    h = jax.nn.relu(_groupnorm(h @ params['fc1_w'].T + params['fc1_b'],
                               params['gn1_g'], params['gn1_b']))
    h = jax.nn.relu(_groupnorm(h @ params['fc3_w'].T + params['fc3_b'],
                               params['gn3_g'], params['gn3_b']))
    h = jax.nn.relu(_groupnorm(h @ params['fc4_w'].T + params['fc4_b'],
                               params['gn4_g'], params['gn4_b']))
    h = h @ params['fc5_w'].T + params['fc5_b']
    position = h[:, :3]
    rot = _rot6d(h[:, 3:])
    return position, rot


# speedup vs baseline: 1.5032x; 1.0319x over previous
import functools

import jax
import jax.numpy as jnp
from jax.experimental import pallas as pl
from jax.experimental.pallas import tpu as pltpu

_F32 = jnp.float32


def _fps_body(S, xyz_ref, idx_ref, dist_ref):
    G, _, N = xyz_ref.shape
    x = xyz_ref[:, 0, :]
    y = xyz_ref[:, 1, :]
    z = xyz_ref[:, 2, :]
    lane = jax.lax.broadcasted_iota(jnp.int32, (G, N), 1)
    s_lane = jax.lax.broadcasted_iota(jnp.int32, (G, S), 1)
    dist_ref[...] = jnp.full((G, N), 1e10, _F32)

    def step(i, far):
        idx_ref[...] = jnp.where(s_lane == i, far, idx_ref[...])
        m = lane == far
        cx = jnp.sum(jnp.where(m, x, 0.0), axis=1, keepdims=True)
        cy = jnp.sum(jnp.where(m, y, 0.0), axis=1, keepdims=True)
        cz = jnp.sum(jnp.where(m, z, 0.0), axis=1, keepdims=True)
        dx = x - cx
        dy = y - cy
        dz = z - cz
        d = dx * dx + dy * dy + dz * dz
        dist = jnp.minimum(dist_ref[...], d)
        dist_ref[...] = dist
        maxv = jnp.max(dist, axis=1, keepdims=True)
        nxt = jnp.min(jnp.where(dist == maxv, lane, N), axis=1, keepdims=True)
        return nxt.astype(jnp.int32)

    jax.lax.fori_loop(0, S, step, jnp.zeros((G, 1), jnp.int32))


def _fps_pallas_raw(xyz_cn, S):
    B, _, N = xyz_cn.shape
    return pl.pallas_call(
        functools.partial(_fps_body, S),
        out_shape=jax.ShapeDtypeStruct((B, S), jnp.int32),
        scratch_shapes=[pltpu.VMEM((B, N), _F32)],
    )(xyz_cn)


def _fps2_body(S2, xyz_ref, idx_ref, out_ref, dist_ref):
    B, _, N = xyz_ref.shape
    S1 = idx_ref.shape[1]
    x = xyz_ref[:, 0, :]
    y = xyz_ref[:, 1, :]
    z = xyz_ref[:, 2, :]
    iota3 = jax.lax.broadcasted_iota(jnp.int32, (B, S1, N), 2)
    oh = idx_ref[...] == iota3
    xs = jnp.sum(jnp.where(oh, x[:, None, :], 0.0), axis=2)
    ys = jnp.sum(jnp.where(oh, y[:, None, :], 0.0), axis=2)
    zs = jnp.sum(jnp.where(oh, z[:, None, :], 0.0), axis=2)

    lane = jax.lax.broadcasted_iota(jnp.int32, (B, S1), 1)
    s_lane = jax.lax.broadcasted_iota(jnp.int32, (B, S2), 1)
    dist_ref[...] = jnp.full((B, S1), 1e10, _F32)

    def step(i, far):
        out_ref[...] = jnp.where(s_lane == i, far, out_ref[...])
        m = lane == far
        cx = jnp.sum(jnp.where(m, xs, 0.0), axis=1, keepdims=True)
        cy = jnp.sum(jnp.where(m, ys, 0.0), axis=1, keepdims=True)
        cz = jnp.sum(jnp.where(m, zs, 0.0), axis=1, keepdims=True)
        dx = xs - cx
        dy = ys - cy
        dz = zs - cz
        d = dx * dx + dy * dy + dz * dz
        dist = jnp.minimum(dist_ref[...], d)
        dist_ref[...] = dist
        maxv = jnp.max(dist, axis=1, keepdims=True)
        nxt = jnp.min(jnp.where(dist == maxv, lane, S1), axis=1, keepdims=True)
        return nxt.astype(jnp.int32)

    jax.lax.fori_loop(0, S2, step, jnp.zeros((B, 1), jnp.int32))


def _fps2_pallas_raw(xyz_cn, fps1_idx, S2):
    B, _, N = xyz_cn.shape
    S1 = fps1_idx.shape[1]
    idx3 = fps1_idx.reshape(B, S1, 1)
    return pl.pallas_call(
        functools.partial(_fps2_body, S2),
        out_shape=jax.ShapeDtypeStruct((B, S2), jnp.int32),
        scratch_shapes=[pltpu.VMEM((B, S1), _F32)],
    )(xyz_cn, idx3)


def _fps_xla(xyz, npoint):
    b, n, _ = xyz.shape

    def step(carry, _):
        distance, farthest = carry
        centroid = jnp.take_along_axis(xyz, farthest[:, None, None], axis=1)
        dist = jnp.sum((xyz - centroid) ** 2, axis=-1)
        distance = jnp.minimum(distance, dist)
        nxt = jnp.argmax(distance, axis=-1).astype(jnp.int32)
        return (distance, nxt), farthest

    init = (jnp.full((b, n), 1e10, dtype=xyz.dtype),
            jnp.zeros((b,), dtype=jnp.int32))
    _, idxs = jax.lax.scan(step, init, None, length=npoint)
    return jnp.transpose(idxs, (1, 0))


def _square_distance(src, dst):
    d = -2.0 * jnp.einsum('bmc,bnc->bmn', src, dst)
    d = d + jnp.sum(src ** 2, axis=-1)[:, :, None]
    d = d + jnp.sum(dst ** 2, axis=-1)[:, None, :]
    return d


def _compute_density(xyz, bandwidth):
    sqrdists = _square_distance(xyz, xyz)
    g = jnp.exp(-sqrdists / (2.0 * bandwidth * bandwidth)) / (2.5 * bandwidth)
    return jnp.mean(g, axis=-1)


def _knn_point(nsample, xyz, new_xyz):
    sqrdists = _square_distance(new_xyz, xyz)
    _, idx = jax.lax.top_k(-sqrdists, nsample)
    return idx


def _index_points(points, idx):
    bidx = jnp.arange(points.shape[0])[:, None, None]
    return points[bidx, idx]


def _batchnorm(x, g, b, axes, eps=1e-5):
    m = jnp.mean(x, axis=axes, keepdims=True)
    v = jnp.var(x, axis=axes, keepdims=True)
    return (x - m) / jnp.sqrt(v + eps) * g + b


def _groupnorm(x, g, b, eps=1e-5):
    m = jnp.mean(x, axis=-1, keepdims=True)
    v = jnp.var(x, axis=-1, keepdims=True)
    return (x - m) / jnp.sqrt(v + eps) * g + b


def _sa_tail(p, new_points, grouped_xyz_norm, grouped_density, b, S):
    h = jnp.einsum('bskc,oc->bsko', new_points, p['conv_w']) + p['conv_b']
    h = jax.nn.relu(_batchnorm(h, p['bn_g'], p['bn_b'], (0, 1, 2)))
    inv_max = jnp.max(grouped_density, axis=2, keepdims=True)
    h = h * (grouped_density / inv_max)
    w = grouped_xyz_norm
    for i in (1, 2, 3):
        w = jnp.einsum('bskc,oc->bsko', w, p['wn%d_w' % i]) + p['wn%d_b' % i]
        w = jax.nn.relu(_batchnorm(w, p['wn%d_g' % i], p['wn%d_b2' % i], (0, 1, 2)))
    feat = jnp.einsum('bskc,bskw->bscw', h, w).reshape(b, S, -1)
    feat = feat @ p['lin_w'].T + p['lin_b']
    feat = jax.nn.relu(_batchnorm(feat, p['bnl_g'], p['bnl_b'], (0, 1)))
    return feat


def _sa_group(xyz, points, fps_idx, nsample, bandwidth):
    density = _compute_density(xyz, bandwidth)
    inv_density = 1.0 / density
    new_xyz = jnp.take_along_axis(xyz, fps_idx[:, :, None], axis=1)
    idx = _knn_point(nsample, xyz, new_xyz)
    grouped_xyz = _index_points(xyz, idx)
    grouped_xyz_norm = grouped_xyz - new_xyz[:, :, None, :]
    grouped_points = _index_points(points, idx)
    new_points = jnp.concatenate([grouped_xyz_norm, grouped_points], axis=-1)
    grouped_density = _index_points(inv_density[:, :, None], idx)
    return new_xyz, new_points, grouped_xyz_norm, grouped_density


def _sa_forward(p, xyz, points, npoint, nsample, bandwidth, fps_idx):
    b = xyz.shape[0]
    new_xyz, new_points, gxn, gd = _sa_group(
        xyz, points, fps_idx, nsample, bandwidth)
    feat = _sa_tail(p, new_points, gxn, gd, b, npoint)
    return new_xyz, feat


def _sa_forward_group_all(p, xyz, points, bandwidth):
    b = xyz.shape[0]
    density = _compute_density(xyz, bandwidth)
    inv_density = 1.0 / density
    grouped_xyz_norm = xyz[:, None, :, :]
    new_points = jnp.concatenate(
        [grouped_xyz_norm, points[:, None, :, :]], axis=-1)
    grouped_density = inv_density[:, None, :, None]
    feat = _sa_tail(p, new_points, grouped_xyz_norm, grouped_density, b, 1)
    return feat


def _rot6d(o):
    x_raw = o[:, 0:3]
    y_raw = o[:, 3:6]

    def normalize(v):
        mag = jnp.sqrt(jnp.sum(v ** 2, axis=1, keepdims=True))
        return v / jnp.maximum(mag, 1e-8)

    x = normalize(x_raw)
    z = normalize(jnp.cross(x, y_raw))
    y = jnp.cross(z, x)
    return jnp.stack([x, y, z], axis=2)


def kernel(xyz, xyz_goal, gripper_eulers, params):
    b = xyz.shape[0]
    l0_xyz = jnp.transpose(xyz[:, :3, :], (0, 2, 1))
    l0_points = jnp.transpose(xyz, (0, 2, 1))
    g0 = jnp.transpose(xyz_goal, (0, 2, 1))

    fps1_m = _fps_pallas_raw(xyz, 512)
    fps1_g = _fps_pallas_raw(xyz_goal, 512)
    l1x, l1p = _sa_forward(params['sa1'], l0_xyz, l0_points, 512, 32, 0.1,
                           fps1_m)
    g1x, g1p = _sa_forward(params['sa1_g'], g0, g0, 512, 32, 0.1, fps1_g)

    l2x, l2p = _sa_forward(params['sa2'], l1x, l1p, 128, 64, 0.2,
                           _fps2_pallas_raw(xyz, fps1_m, 128))
    g2x, g2p = _sa_forward(params['sa2_g'], g1x, g1p, 128, 64, 0.2,
                           _fps2_pallas_raw(xyz_goal, fps1_g, 128))

    l3p = _sa_forward_group_all(params['sa3'], l2x, l2p, 0.4)
    g3p = _sa_forward_group_all(params['sa3_g'], g2x, g2p, 0.4)
    x = l3p.reshape(b, 256)
    g = g3p.reshape(b, 256)

    gr = jax.nn.relu(_groupnorm(
        gripper_eulers @ params['fc1g_w'].T + params['fc1g_b'],
        params['gn1g_g'], params['gn1g_b']))
    gr = jax.nn.relu(_groupnorm(
        gr @ params['fc2g_w'].T + params['fc2g_b'],
        params['gn2g_g'], params['gn2g_b']))
    h = jnp.concatenate([x, g, gr], axis=-1)
    h = jax.nn.relu(_groupnorm(h @ params['fc1_w'].T + params['fc1_b'],
                               params['gn1_g'], params['gn1_b']))
    h = jax.nn.relu(_groupnorm(h @ params['fc3_w'].T + params['fc3_b'],
                               params['gn3_g'], params['gn3_b']))
    h = jax.nn.relu(_groupnorm(h @ params['fc4_w'].T + params['fc4_b'],
                               params['gn4_g'], params['gn4_b']))
    h = h @ params['fc5_w'].T + params['fc5_b']
    position = h[:, :3]
    rot = _rot6d(h[:, 3:])
    return position, rot


# speedup vs baseline: 1.5037x; 1.0003x over previous
import functools

import jax
import jax.numpy as jnp
from jax.experimental import pallas as pl
from jax.experimental.pallas import tpu as pltpu

_F32 = jnp.float32


def _fps_body(S, xyz_ref, idx_ref, dist_ref):
    G, _, N = xyz_ref.shape
    x = xyz_ref[:, 0, :]
    y = xyz_ref[:, 1, :]
    z = xyz_ref[:, 2, :]
    lane = jax.lax.broadcasted_iota(jnp.int32, (G, N), 1)
    s_lane = jax.lax.broadcasted_iota(jnp.int32, (G, S), 1)
    dist_ref[...] = jnp.full((G, N), 1e10, _F32)

    def step(i, far):
        idx_ref[...] = jnp.where(s_lane == i, far, idx_ref[...])
        m = lane == far
        cx = jnp.sum(jnp.where(m, x, 0.0), axis=1, keepdims=True)
        cy = jnp.sum(jnp.where(m, y, 0.0), axis=1, keepdims=True)
        cz = jnp.sum(jnp.where(m, z, 0.0), axis=1, keepdims=True)
        dx = x - cx
        dy = y - cy
        dz = z - cz
        d = dx * dx + dy * dy + dz * dz
        dist = jnp.minimum(dist_ref[...], d)
        dist_ref[...] = dist
        maxv = jnp.max(dist, axis=1, keepdims=True)
        nxt = jnp.min(jnp.where(dist == maxv, lane, N), axis=1, keepdims=True)
        return nxt.astype(jnp.int32)

    jax.lax.fori_loop(0, S, step, jnp.zeros((G, 1), jnp.int32))


def _fps_pallas_raw(xyz_cn, S):
    B, _, N = xyz_cn.shape
    return pl.pallas_call(
        functools.partial(_fps_body, S),
        out_shape=jax.ShapeDtypeStruct((B, S), jnp.int32),
        scratch_shapes=[pltpu.VMEM((B, N), _F32)],
    )(xyz_cn)


def _fps2_body(S2, xyz_ref, idx_ref, out_ref, dist_ref):
    B, _, N = xyz_ref.shape
    S1 = idx_ref.shape[1]
    x = xyz_ref[:, 0, :]
    y = xyz_ref[:, 1, :]
    z = xyz_ref[:, 2, :]
    iota3 = jax.lax.broadcasted_iota(jnp.int32, (B, S1, N), 2)
    oh = idx_ref[...] == iota3
    xs = jnp.sum(jnp.where(oh, x[:, None, :], 0.0), axis=2)
    ys = jnp.sum(jnp.where(oh, y[:, None, :], 0.0), axis=2)
    zs = jnp.sum(jnp.where(oh, z[:, None, :], 0.0), axis=2)

    lane = jax.lax.broadcasted_iota(jnp.int32, (B, S1), 1)
    s_lane = jax.lax.broadcasted_iota(jnp.int32, (B, S2), 1)
    dist_ref[...] = jnp.full((B, S1), 1e10, _F32)

    def step(i, far):
        out_ref[...] = jnp.where(s_lane == i, far, out_ref[...])
        m = lane == far
        cx = jnp.sum(jnp.where(m, xs, 0.0), axis=1, keepdims=True)
        cy = jnp.sum(jnp.where(m, ys, 0.0), axis=1, keepdims=True)
        cz = jnp.sum(jnp.where(m, zs, 0.0), axis=1, keepdims=True)
        dx = xs - cx
        dy = ys - cy
        dz = zs - cz
        d = dx * dx + dy * dy + dz * dz
        dist = jnp.minimum(dist_ref[...], d)
        dist_ref[...] = dist
        maxv = jnp.max(dist, axis=1, keepdims=True)
        nxt = jnp.min(jnp.where(dist == maxv, lane, S1), axis=1, keepdims=True)
        return nxt.astype(jnp.int32)

    jax.lax.fori_loop(0, S2, step, jnp.zeros((B, 1), jnp.int32))


def _fps2_pallas_raw(xyz_cn, fps1_idx, S2):
    B, _, N = xyz_cn.shape
    S1 = fps1_idx.shape[1]
    idx3 = fps1_idx.reshape(B, S1, 1)
    return pl.pallas_call(
        functools.partial(_fps2_body, S2),
        out_shape=jax.ShapeDtypeStruct((B, S2), jnp.int32),
        scratch_shapes=[pltpu.VMEM((B, S1), _F32)],
    )(xyz_cn, idx3)


def _square_distance(src, dst):
    d = -2.0 * jnp.einsum('bmc,bnc->bmn', src, dst)
    d = d + jnp.sum(src ** 2, axis=-1)[:, :, None]
    d = d + jnp.sum(dst ** 2, axis=-1)[:, None, :]
    return d


def _compute_density(xyz, bandwidth):
    sqrdists = _square_distance(xyz, xyz)
    g = jnp.exp(-sqrdists / (2.0 * bandwidth * bandwidth)) / (2.5 * bandwidth)
    return jnp.mean(g, axis=-1)


def _knn_point(nsample, xyz, new_xyz):
    sqrdists = _square_distance(new_xyz, xyz)
    _, idx = jax.lax.top_k(-sqrdists, nsample)
    return idx


def _index_points(points, idx):
    bidx = jnp.arange(points.shape[0])[:, None, None]
    return points[bidx, idx]


def _batchnorm(x, g, b, axes, eps=1e-5):
    m = jnp.mean(x, axis=axes, keepdims=True)
    v = jnp.var(x, axis=axes, keepdims=True)
    return (x - m) / jnp.sqrt(v + eps) * g + b


def _groupnorm(x, g, b, eps=1e-5):
    m = jnp.mean(x, axis=-1, keepdims=True)
    v = jnp.var(x, axis=-1, keepdims=True)
    return (x - m) / jnp.sqrt(v + eps) * g + b


def _sa_tail(p, new_points, grouped_xyz_norm, grouped_density, b, S):
    h = jnp.einsum('bskc,oc->bsko', new_points, p['conv_w']) + p['conv_b']
    h = jax.nn.relu(_batchnorm(h, p['bn_g'], p['bn_b'], (0, 1, 2)))
    inv_max = jnp.max(grouped_density, axis=2, keepdims=True)
    h = h * (grouped_density / inv_max)
    w = grouped_xyz_norm
    for i in (1, 2, 3):
        w = jnp.einsum('bskc,oc->bsko', w, p['wn%d_w' % i]) + p['wn%d_b' % i]
        w = jax.nn.relu(_batchnorm(w, p['wn%d_g' % i], p['wn%d_b2' % i], (0, 1, 2)))
    feat = jnp.einsum('bskc,bskw->bscw', h, w).reshape(b, S, -1)
    feat = feat @ p['lin_w'].T + p['lin_b']
    feat = jax.nn.relu(_batchnorm(feat, p['bnl_g'], p['bnl_b'], (0, 1)))
    return feat


def _sa_group(xyz, points, fps_idx, nsample, bandwidth):
    density = _compute_density(xyz, bandwidth)
    inv_density = 1.0 / density
    new_xyz = jnp.take_along_axis(xyz, fps_idx[:, :, None], axis=1)
    idx = _knn_point(nsample, xyz, new_xyz)
    grouped_xyz = _index_points(xyz, idx)
    grouped_xyz_norm = grouped_xyz - new_xyz[:, :, None, :]
    grouped_points = _index_points(points, idx)
    new_points = jnp.concatenate([grouped_xyz_norm, grouped_points], axis=-1)
    grouped_density = _index_points(inv_density[:, :, None], idx)
    return new_xyz, new_points, grouped_xyz_norm, grouped_density


def _sa_forward(p, xyz, points, npoint, nsample, bandwidth, fps_idx):
    b = xyz.shape[0]
    new_xyz, new_points, gxn, gd = _sa_group(
        xyz, points, fps_idx, nsample, bandwidth)
    feat = _sa_tail(p, new_points, gxn, gd, b, npoint)
    return new_xyz, feat


def _sa_forward_group_all(p, xyz, points, bandwidth):
    b = xyz.shape[0]
    density = _compute_density(xyz, bandwidth)
    inv_density = 1.0 / density
    grouped_xyz_norm = xyz[:, None, :, :]
    new_points = jnp.concatenate(
        [grouped_xyz_norm, points[:, None, :, :]], axis=-1)
    grouped_density = inv_density[:, None, :, None]
    feat = _sa_tail(p, new_points, grouped_xyz_norm, grouped_density, b, 1)
    return feat


def _rot6d(o):
    x_raw = o[:, 0:3]
    y_raw = o[:, 3:6]

    def normalize(v):
        mag = jnp.sqrt(jnp.sum(v ** 2, axis=1, keepdims=True))
        return v / jnp.maximum(mag, 1e-8)

    x = normalize(x_raw)
    z = normalize(jnp.cross(x, y_raw))
    y = jnp.cross(z, x)
    return jnp.stack([x, y, z], axis=2)


def kernel(xyz, xyz_goal, gripper_eulers, params):
    b = xyz.shape[0]
    l0_xyz = jnp.transpose(xyz[:, :3, :], (0, 2, 1))
    l0_points = jnp.transpose(xyz, (0, 2, 1))
    g0 = jnp.transpose(xyz_goal, (0, 2, 1))

    fps1_m = _fps_pallas_raw(xyz, 512)
    fps1_g = _fps_pallas_raw(xyz_goal, 512)
    l1x, l1p = _sa_forward(params['sa1'], l0_xyz, l0_points, 512, 32, 0.1,
                           fps1_m)
    g1x, g1p = _sa_forward(params['sa1_g'], g0, g0, 512, 32, 0.1, fps1_g)

    l2x, l2p = _sa_forward(params['sa2'], l1x, l1p, 128, 64, 0.2,
                           _fps2_pallas_raw(xyz, fps1_m, 128))
    g2x, g2p = _sa_forward(params['sa2_g'], g1x, g1p, 128, 64, 0.2,
                           _fps2_pallas_raw(xyz_goal, fps1_g, 128))

    l3p = _sa_forward_group_all(params['sa3'], l2x, l2p, 0.4)
    g3p = _sa_forward_group_all(params['sa3_g'], g2x, g2p, 0.4)
    x = l3p.reshape(b, 256)
    g = g3p.reshape(b, 256)

    gr = jax.nn.relu(_groupnorm(
        gripper_eulers @ params['fc1g_w'].T + params['fc1g_b'],
        params['gn1g_g'], params['gn1g_b']))
    gr = jax.nn.relu(_groupnorm(
        gr @ params['fc2g_w'].T + params['fc2g_b'],
        params['gn2g_g'], params['gn2g_b']))
    h = jnp.concatenate([x, g, gr], axis=-1)
    h = jax.nn.relu(_groupnorm(h @ params['fc1_w'].T + params['fc1_b'],
                               params['gn1_g'], params['gn1_b']))
    h = jax.nn.relu(_groupnorm(h @ params['fc3_w'].T + params['fc3_b'],
                               params['gn3_g'], params['gn3_b']))
    h = jax.nn.relu(_groupnorm(h @ params['fc4_w'].T + params['fc4_b'],
                               params['gn4_g'], params['gn4_b']))
    h = h @ params['fc5_w'].T + params['fc5_b']
    position = h[:, :3]
    rot = _rot6d(h[:, 3:])
    return position, rot
